# baseline (speedup 1.0000x reference)
import jax
import jax.numpy as jnp
from jax.experimental import pallas as pl
from jax.experimental.pallas import tpu as pltpu

_NQ = 4


def _mf_kernel(uid_ref,
               user_hbm, label_hbm,
               item_ref,
               pid_ref, nid_ref,
               user_out, pos_out, neg_out, com_out,
               bl_buf, sems):
    c = pl.program_id(0)
    bt, num_items = bl_buf.shape
    base = c * bt
    qs = bt // _NQ

    for q in range(_NQ):
        for i in range(qs):
            j = q * qs + i
            u = uid_ref[base + j]
            pltpu.make_async_copy(
                label_hbm.at[pl.ds(u, 1), :], bl_buf.at[pl.ds(j, 1), :],
                sems.at[q]).start(priority=q % 2)

    for j in range(bt):
        u = uid_ref[base + j]
        pltpu.make_async_copy(
            user_hbm.at[pl.ds(u, 1), :], user_out.at[pl.ds(j, 1), :],
            sems.at[_NQ + j % 2]).start(priority=j % 2)

    item = item_ref[...]

    lane = jax.lax.broadcasted_iota(jnp.int32, (bt, num_items), 1)
    oh_pos = (pid_ref[...] == lane).astype(jnp.float32)
    oh_neg = (nid_ref[...] == lane).astype(jnp.float32)
    pos_out[...] = jnp.dot(oh_pos, item, preferred_element_type=jnp.float32)
    neg_out[...] = jnp.dot(oh_neg, item, preferred_element_type=jnp.float32)

    for q in range(_NQ):
        pltpu.make_async_copy(
            label_hbm.at[pl.ds(0, qs), :], bl_buf.at[pl.ds(0, qs), :],
            sems.at[q]).wait()
        bl = bl_buf[pl.ds(q * qs, qs), :]
        acc = jnp.dot(bl, item, preferred_element_type=jnp.float32)
        num = jnp.sum(bl, axis=1, keepdims=True)
        com_out[pl.ds(q * qs, qs), :] = acc / jnp.where(num > 0.0, num, 1.0)

    h = bt // 2
    for s in range(2):
        pltpu.make_async_copy(
            user_hbm.at[pl.ds(0, h), :], user_out.at[pl.ds(0, h), :],
            sems.at[_NQ + s]).wait()


def kernel(user_id, pos_id, neg_id, user_table, item_table, train_label):
    B = user_id.shape[0]
    num_users, dim = user_table.shape
    num_items = item_table.shape[0]

    align = 8 * _NQ
    bt = align * pl.cdiv(B, 2 * align)
    Bp = 2 * bt
    pad = Bp - B

    uid = user_id.astype(jnp.int32)
    pid = pos_id.astype(jnp.int32)
    nid = neg_id.astype(jnp.int32)
    if pad:
        uid = jnp.pad(uid, (0, pad))
        pid = jnp.pad(pid, (0, pad))
        nid = jnp.pad(nid, (0, pad))
    pid2 = pid.reshape(Bp, 1)
    nid2 = nid.reshape(Bp, 1)

    grid_spec = pltpu.PrefetchScalarGridSpec(
        num_scalar_prefetch=1,
        grid=(2,),
        in_specs=[
            pl.BlockSpec(memory_space=pl.ANY),
            pl.BlockSpec(memory_space=pl.ANY),
            pl.BlockSpec((num_items, dim), lambda c, uid: (0, 0)),
            pl.BlockSpec((bt, 1), lambda c, uid: (c, 0)),
            pl.BlockSpec((bt, 1), lambda c, uid: (c, 0)),
        ],
        out_specs=[pl.BlockSpec((bt, dim), lambda c, uid: (c, 0))] * 4,
        scratch_shapes=[
            pltpu.VMEM((bt, num_items), jnp.float32),
            pltpu.SemaphoreType.DMA((_NQ + 2,)),
        ],
    )

    outs = pl.pallas_call(
        _mf_kernel,
        out_shape=[jax.ShapeDtypeStruct((Bp, dim), jnp.float32)] * 4,
        grid_spec=grid_spec,
        compiler_params=pltpu.CompilerParams(
            dimension_semantics=("parallel",),
            vmem_limit_bytes=60 * 1024 * 1024),
    )(uid,
      user_table.astype(jnp.float32),
      train_label.astype(jnp.float32),
      item_table.astype(jnp.float32),
      pid2, nid2)

    if pad:
        outs = [o[:B] for o in outs]
    return tuple(outs)


# speedup vs baseline: 1.1889x; 1.1889x over previous
import jax
import jax.numpy as jnp
from jax.experimental import pallas as pl
from jax.experimental.pallas import tpu as pltpu

_NQ = 4


def _mf_kernel(uid_ref,
               user_hbm, label_hbm,
               item_ref,
               pid_ref, nid_ref,
               user_out, pos_out, neg_out, com_out,
               bl_buf, sems):
    c = pl.program_id(0)
    bt, num_items = bl_buf.shape
    base = c * bt
    qs = bt // _NQ

    for q in range(_NQ):
        for i in range(qs):
            j = q * qs + i
            u = uid_ref[base + j]
            pltpu.make_async_copy(
                label_hbm.at[pl.ds(u, 1), :], bl_buf.at[pl.ds(j, 1), :],
                sems.at[q]).start(priority=q % 2)

    for j in range(bt):
        u = uid_ref[base + j]
        pltpu.make_async_copy(
            user_hbm.at[pl.ds(u, 1), :], user_out.at[pl.ds(j, 1), :],
            sems.at[_NQ + j % 2]).start(priority=j % 2)

    item = item_ref[...]

    laneT = jax.lax.broadcasted_iota(jnp.int32, (num_items, bt), 0)
    ohT_pos = (pid_ref[...] == laneT).astype(jnp.float32)
    ohT_neg = (nid_ref[...] == laneT).astype(jnp.float32)
    dn = (((0,), (0,)), ((), ()))
    pos_out[...] = jax.lax.dot_general(ohT_pos, item, dimension_numbers=dn,
                                       preferred_element_type=jnp.float32)
    neg_out[...] = jax.lax.dot_general(ohT_neg, item, dimension_numbers=dn,
                                       preferred_element_type=jnp.float32)

    for q in range(_NQ):
        pltpu.make_async_copy(
            label_hbm.at[pl.ds(0, qs), :], bl_buf.at[pl.ds(0, qs), :],
            sems.at[q]).wait()
        bl = bl_buf[pl.ds(q * qs, qs), :]
        acc = jnp.dot(bl, item, preferred_element_type=jnp.float32)
        num = jnp.sum(bl, axis=1, keepdims=True)
        com_out[pl.ds(q * qs, qs), :] = acc / jnp.where(num > 0.0, num, 1.0)

    h = bt // 2
    for s in range(2):
        pltpu.make_async_copy(
            user_hbm.at[pl.ds(0, h), :], user_out.at[pl.ds(0, h), :],
            sems.at[_NQ + s]).wait()


def kernel(user_id, pos_id, neg_id, user_table, item_table, train_label):
    B = user_id.shape[0]
    num_users, dim = user_table.shape
    num_items = item_table.shape[0]

    align = 8 * _NQ
    bt = align * pl.cdiv(B, 2 * align)
    Bp = 2 * bt
    pad = Bp - B

    uid = user_id.astype(jnp.int32)
    pid = pos_id.astype(jnp.int32)
    nid = neg_id.astype(jnp.int32)
    if pad:
        uid = jnp.pad(uid, (0, pad))
        pid = jnp.pad(pid, (0, pad))
        nid = jnp.pad(nid, (0, pad))
    pid2 = pid.reshape(1, Bp)
    nid2 = nid.reshape(1, Bp)

    grid_spec = pltpu.PrefetchScalarGridSpec(
        num_scalar_prefetch=1,
        grid=(2,),
        in_specs=[
            pl.BlockSpec(memory_space=pl.ANY),
            pl.BlockSpec(memory_space=pl.ANY),
            pl.BlockSpec((num_items, dim), lambda c, uid: (0, 0)),
            pl.BlockSpec((1, bt), lambda c, uid: (0, c)),
            pl.BlockSpec((1, bt), lambda c, uid: (0, c)),
        ],
        out_specs=[pl.BlockSpec((bt, dim), lambda c, uid: (c, 0))] * 4,
        scratch_shapes=[
            pltpu.VMEM((bt, num_items), jnp.float32),
            pltpu.SemaphoreType.DMA((_NQ + 2,)),
        ],
    )

    outs = pl.pallas_call(
        _mf_kernel,
        out_shape=[jax.ShapeDtypeStruct((Bp, dim), jnp.float32)] * 4,
        grid_spec=grid_spec,
        compiler_params=pltpu.CompilerParams(
            dimension_semantics=("parallel",),
            vmem_limit_bytes=60 * 1024 * 1024),
    )(uid,
      user_table.astype(jnp.float32),
      train_label.astype(jnp.float32),
      item_table.astype(jnp.float32),
      pid2, nid2)

    if pad:
        outs = [o[:B] for o in outs]
    return tuple(outs)


# speedup vs baseline: 1.1898x; 1.0008x over previous
import jax
import jax.numpy as jnp
from jax.experimental import pallas as pl
from jax.experimental.pallas import tpu as pltpu

_NQ = 4


def _mf_kernel(uid_ref,
               user_hbm, label_hbm,
               item_ref,
               pid_ref, nid_ref,
               user_out, pos_out, neg_out, com_out,
               bl_buf, sems):
    c = pl.program_id(0)
    bt, num_items = bl_buf.shape
    base = c * bt
    qs = bt // _NQ

    for q in range(_NQ):
        for i in range(qs):
            j = q * qs + i
            u = uid_ref[base + j]
            pltpu.make_async_copy(
                label_hbm.at[pl.ds(u, 1), :], bl_buf.at[pl.ds(j, 1), :],
                sems.at[q]).start(priority=q % 2)

    for j in range(bt):
        u = uid_ref[base + j]
        pltpu.make_async_copy(
            user_hbm.at[pl.ds(u, 1), :], user_out.at[pl.ds(j, 1), :],
            sems.at[_NQ + j % 2]).start(priority=j % 2)

    item = item_ref[...]

    laneT = jax.lax.broadcasted_iota(jnp.int32, (num_items, bt), 0)
    ohT_pos = (pid_ref[...] == laneT).astype(jnp.float32)
    ohT_neg = (nid_ref[...] == laneT).astype(jnp.float32)
    dn = (((0,), (0,)), ((), ()))
    pos_out[...] = jax.lax.dot_general(ohT_pos, item, dimension_numbers=dn,
                                       preferred_element_type=jnp.float32)
    neg_out[...] = jax.lax.dot_general(ohT_neg, item, dimension_numbers=dn,
                                       preferred_element_type=jnp.float32)

    item_bf = item.astype(jnp.bfloat16)
    for q in range(_NQ):
        pltpu.make_async_copy(
            label_hbm.at[pl.ds(0, qs), :], bl_buf.at[pl.ds(0, qs), :],
            sems.at[q]).wait()
        bl = bl_buf[pl.ds(q * qs, qs), :]
        acc = jnp.dot(bl.astype(jnp.bfloat16), item_bf,
                      preferred_element_type=jnp.float32)
        num = jnp.sum(bl, axis=1, keepdims=True)
        com_out[pl.ds(q * qs, qs), :] = acc / jnp.where(num > 0.0, num, 1.0)

    h = bt // 2
    for s in range(2):
        pltpu.make_async_copy(
            user_hbm.at[pl.ds(0, h), :], user_out.at[pl.ds(0, h), :],
            sems.at[_NQ + s]).wait()


def kernel(user_id, pos_id, neg_id, user_table, item_table, train_label):
    B = user_id.shape[0]
    num_users, dim = user_table.shape
    num_items = item_table.shape[0]

    align = 8 * _NQ
    bt = align * pl.cdiv(B, 2 * align)
    Bp = 2 * bt
    pad = Bp - B

    uid = user_id.astype(jnp.int32)
    pid = pos_id.astype(jnp.int32)
    nid = neg_id.astype(jnp.int32)
    if pad:
        uid = jnp.pad(uid, (0, pad))
        pid = jnp.pad(pid, (0, pad))
        nid = jnp.pad(nid, (0, pad))
    pid2 = pid.reshape(1, Bp)
    nid2 = nid.reshape(1, Bp)

    grid_spec = pltpu.PrefetchScalarGridSpec(
        num_scalar_prefetch=1,
        grid=(2,),
        in_specs=[
            pl.BlockSpec(memory_space=pl.ANY),
            pl.BlockSpec(memory_space=pl.ANY),
            pl.BlockSpec((num_items, dim), lambda c, uid: (0, 0)),
            pl.BlockSpec((1, bt), lambda c, uid: (0, c)),
            pl.BlockSpec((1, bt), lambda c, uid: (0, c)),
        ],
        out_specs=[pl.BlockSpec((bt, dim), lambda c, uid: (c, 0))] * 4,
        scratch_shapes=[
            pltpu.VMEM((bt, num_items), jnp.float32),
            pltpu.SemaphoreType.DMA((_NQ + 2,)),
        ],
    )

    outs = pl.pallas_call(
        _mf_kernel,
        out_shape=[jax.ShapeDtypeStruct((Bp, dim), jnp.float32)] * 4,
        grid_spec=grid_spec,
        compiler_params=pltpu.CompilerParams(
            dimension_semantics=("parallel",),
            vmem_limit_bytes=60 * 1024 * 1024),
    )(uid,
      user_table.astype(jnp.float32),
      train_label.astype(jnp.float32),
      item_table.astype(jnp.float32),
      pid2, nid2)

    if pad:
        outs = [o[:B] for o in outs]
    return tuple(outs)
